# Initial kernel scaffold; baseline (speedup 1.0000x reference)
import jax
import jax.numpy as jnp
from jax.experimental import pallas as pl


def kernel(x, W_base, b_base, W_router, lora_A, lora_B):
    raise NotImplementedError("write your pallas kernel here")


# speedup vs baseline: 4.8202x; 4.8202x over previous
import jax
import jax.numpy as jnp
from jax.experimental import pallas as pl
from jax.experimental.pallas import tpu as pltpu

D_MODEL = 2048
N_TOK = 8192
N_EXPERTS = 8
TOP_K = 2
RANK = 16
SCALING = 32.0 / 16.0

BN = 1024
BD = 512


def _fused_kernel(x_ref, wb_ref, b_ref, wr_ref, aall_ref, ball_ref,
                  out_ref, aw_ref):
    @pl.when(pl.program_id(1) == 0)
    def _router_and_bottleneck():
        xb = x_ref[...]
        lg = jax.lax.dot_general(
            xb, wr_ref[...], (((1,), (1,)), ((), ())),
            preferred_element_type=jnp.float32)
        eid = jax.lax.broadcasted_iota(jnp.int32, lg.shape, 1) // RANK
        m1 = jnp.max(lg, axis=1, keepdims=True)
        i1 = jnp.min(jnp.where(lg == m1, eid, N_EXPERTS), axis=1,
                     keepdims=True)
        lg2 = jnp.where(eid == i1, -jnp.inf, lg)
        m2 = jnp.max(lg2, axis=1, keepdims=True)
        i2 = jnp.min(jnp.where(lg2 == m2, eid, N_EXPERTS), axis=1,
                     keepdims=True)
        w1 = 1.0 / (1.0 + jnp.exp(m2 - m1))
        w2 = 1.0 - w1
        wexp = jnp.where(eid == i1, w1, jnp.where(eid == i2, w2, 0.0))
        a = jax.lax.dot_general(
            xb, aall_ref[...], (((1,), (1,)), ((), ())),
            preferred_element_type=jnp.float32)
        aw_ref[...] = a * (wexp * SCALING)

    acc = jax.lax.dot_general(
        x_ref[...], wb_ref[...], (((1,), (1,)), ((), ())),
        preferred_element_type=jnp.float32)
    acc = acc + jnp.dot(aw_ref[...], ball_ref[...],
                        preferred_element_type=jnp.float32)
    out_ref[...] = acc + b_ref[...]


def kernel(x, W_base, b_base, W_router, lora_A, lora_B):
    wr_rep = jnp.repeat(W_router, RANK, axis=0)
    a_all = lora_A.reshape(N_EXPERTS * RANK, D_MODEL)
    b_all = lora_B.transpose(0, 2, 1).reshape(N_EXPERTS * RANK, D_MODEL)
    b2 = b_base.reshape(1, D_MODEL)

    n_tiles = N_TOK // BN
    d_tiles = D_MODEL // BD
    return pl.pallas_call(
        _fused_kernel,
        grid=(n_tiles, d_tiles),
        in_specs=[
            pl.BlockSpec((BN, D_MODEL), lambda n, d: (n, 0)),
            pl.BlockSpec((BD, D_MODEL), lambda n, d: (d, 0)),
            pl.BlockSpec((1, BD), lambda n, d: (0, d)),
            pl.BlockSpec((N_EXPERTS * RANK, D_MODEL),
                         lambda n, d: (0, 0)),
            pl.BlockSpec((N_EXPERTS * RANK, D_MODEL),
                         lambda n, d: (0, 0)),
            pl.BlockSpec((N_EXPERTS * RANK, BD), lambda n, d: (0, d)),
        ],
        out_specs=pl.BlockSpec((BN, BD), lambda n, d: (n, d)),
        out_shape=jax.ShapeDtypeStruct((N_TOK, D_MODEL), jnp.float32),
        scratch_shapes=[pltpu.VMEM((BN, N_EXPERTS * RANK), jnp.float32)],
        compiler_params=pltpu.CompilerParams(
            dimension_semantics=("parallel", "arbitrary")),
    )(x, W_base, b2, wr_rep, a_all, b_all)
